# initial kernel scaffold (unmeasured)
import jax
import jax.numpy as jnp
from jax import lax
from jax.experimental import pallas as pl
from jax.experimental.pallas import tpu as pltpu


def kernel(
    x,
):
    def body(*refs):
        pass

    out_shape = jax.ShapeDtypeStruct(..., jnp.float32)
    return pl.pallas_call(body, out_shape=out_shape)(...)


# baseline (device time: 436332 ns/iter reference)
import jax
import jax.numpy as jnp
from jax import lax
from jax.experimental import pallas as pl
from jax.experimental.pallas import tpu as pltpu

N_CHUNKS = 8


def kernel(x):
    x = x.astype(jnp.bfloat16)
    m, n = x.shape
    rows = m // N_CHUNKS

    def body(x_ref, out_ref, xbuf, vx, vp, vs,
             lsem_a, lsem_b, lsem_c, send_x, recv_x, send_y, recv_y):
        my_x = lax.axis_index("x")
        my_y = lax.axis_index("y")
        x_peer = (1 - my_x, my_y)
        y_peer = (my_x, 1 - my_y)

        barrier = pltpu.get_barrier_semaphore()
        for nbr in (x_peer, y_peer):
            pl.semaphore_signal(
                barrier, inc=1, device_id=nbr,
                device_id_type=pl.DeviceIdType.MESH,
            )
        pl.semaphore_wait(barrier, 2)

        rdma_x = pltpu.make_async_remote_copy(
            src_ref=x_ref,
            dst_ref=xbuf,
            send_sem=send_x,
            recv_sem=recv_x,
            device_id=x_peer,
            device_id_type=pl.DeviceIdType.MESH,
        )
        rdma_x.start()
        rdma_x.wait()

        for k in range(N_CHUNKS):
            r0 = k * rows
            cp_x = pltpu.make_async_copy(
                x_ref.at[pl.ds(r0, rows), :], vx, lsem_a)
            cp_p = pltpu.make_async_copy(
                xbuf.at[pl.ds(r0, rows), :], vp, lsem_b)
            cp_x.start()
            cp_p.start()
            cp_x.wait()
            cp_p.wait()
            vs[:, :] = vx[:, :] + vp[:, :]
            cp_o = pltpu.make_async_copy(
                vs, out_ref.at[pl.ds(r0, rows), pl.ds(my_y * n, n)], lsem_c)
            cp_o.start()
            cp_o.wait()

        rdma_y = pltpu.make_async_remote_copy(
            src_ref=out_ref.at[:, pl.ds(my_y * n, n)],
            dst_ref=out_ref.at[:, pl.ds(my_y * n, n)],
            send_sem=send_y,
            recv_sem=recv_y,
            device_id=y_peer,
            device_id_type=pl.DeviceIdType.MESH,
        )
        rdma_y.start()
        rdma_y.wait()

    out, _ = pl.pallas_call(
        body,
        out_shape=[
            jax.ShapeDtypeStruct((m, 2 * n), jnp.bfloat16),
            jax.ShapeDtypeStruct((m, n), jnp.bfloat16),
        ],
        in_specs=[pl.BlockSpec(memory_space=pl.ANY)],
        out_specs=[
            pl.BlockSpec(memory_space=pl.ANY),
            pl.BlockSpec(memory_space=pl.ANY),
        ],
        scratch_shapes=[
            pltpu.VMEM((rows, n), jnp.bfloat16),
            pltpu.VMEM((rows, n), jnp.bfloat16),
            pltpu.VMEM((rows, n), jnp.bfloat16),
            pltpu.SemaphoreType.DMA,
            pltpu.SemaphoreType.DMA,
            pltpu.SemaphoreType.DMA,
            pltpu.SemaphoreType.DMA,
            pltpu.SemaphoreType.DMA,
            pltpu.SemaphoreType.DMA,
            pltpu.SemaphoreType.DMA,
        ],
        compiler_params=pltpu.CompilerParams(collective_id=0),
    )(x)
    return out


# device time: 253081 ns/iter; 1.7241x vs baseline; 1.7241x over previous
import jax
import jax.numpy as jnp
from jax import lax
from jax.experimental import pallas as pl
from jax.experimental.pallas import tpu as pltpu

N_CHUNKS = 8


def kernel(x):
    x = x.astype(jnp.bfloat16)
    m, n = x.shape
    rows = m // N_CHUNKS

    def body(x_ref, out_ref, xbuf, vx, vp, vs,
             lsem_a, lsem_b, lsem_c, send_x, recv_x, send_y, recv_y):
        my_x = lax.axis_index("x")
        my_y = lax.axis_index("y")
        x_peer = (1 - my_x, my_y)
        y_peer = (my_x, 1 - my_y)

        barrier = pltpu.get_barrier_semaphore()
        for nbr in (x_peer, y_peer):
            pl.semaphore_signal(
                barrier, inc=1, device_id=nbr,
                device_id_type=pl.DeviceIdType.MESH,
            )
        pl.semaphore_wait(barrier, 2)

        x_rdmas = []
        for k in range(N_CHUNKS):
            r0 = k * rows
            r = pltpu.make_async_remote_copy(
                src_ref=x_ref.at[pl.ds(r0, rows), :],
                dst_ref=xbuf.at[pl.ds(r0, rows), :],
                send_sem=send_x.at[k],
                recv_sem=recv_x.at[k],
                device_id=x_peer,
                device_id_type=pl.DeviceIdType.MESH,
            )
            r.start()
            x_rdmas.append(r)

        y_rdmas = []
        for k in range(N_CHUNKS):
            r0 = k * rows
            x_rdmas[k].wait_recv()
            cp_x = pltpu.make_async_copy(
                x_ref.at[pl.ds(r0, rows), :], vx, lsem_a)
            cp_p = pltpu.make_async_copy(
                xbuf.at[pl.ds(r0, rows), :], vp, lsem_b)
            cp_x.start()
            cp_p.start()
            cp_x.wait()
            cp_p.wait()
            vs[:, :] = vx[:, :] + vp[:, :]
            cp_o = pltpu.make_async_copy(
                vs, out_ref.at[pl.ds(r0, rows), pl.ds(my_y * n, n)], lsem_c)
            cp_o.start()
            cp_o.wait()
            ry = pltpu.make_async_remote_copy(
                src_ref=out_ref.at[pl.ds(r0, rows), pl.ds(my_y * n, n)],
                dst_ref=out_ref.at[pl.ds(r0, rows), pl.ds(my_y * n, n)],
                send_sem=send_y.at[k],
                recv_sem=recv_y.at[k],
                device_id=y_peer,
                device_id_type=pl.DeviceIdType.MESH,
            )
            ry.start()
            y_rdmas.append(ry)

        for k in range(N_CHUNKS):
            x_rdmas[k].wait_send()
            y_rdmas[k].wait_send()
            y_rdmas[k].wait_recv()

    out, _ = pl.pallas_call(
        body,
        out_shape=[
            jax.ShapeDtypeStruct((m, 2 * n), jnp.bfloat16),
            jax.ShapeDtypeStruct((m, n), jnp.bfloat16),
        ],
        in_specs=[pl.BlockSpec(memory_space=pl.ANY)],
        out_specs=[
            pl.BlockSpec(memory_space=pl.ANY),
            pl.BlockSpec(memory_space=pl.ANY),
        ],
        scratch_shapes=[
            pltpu.VMEM((rows, n), jnp.bfloat16),
            pltpu.VMEM((rows, n), jnp.bfloat16),
            pltpu.VMEM((rows, n), jnp.bfloat16),
            pltpu.SemaphoreType.DMA,
            pltpu.SemaphoreType.DMA,
            pltpu.SemaphoreType.DMA,
            pltpu.SemaphoreType.DMA((N_CHUNKS,)),
            pltpu.SemaphoreType.DMA((N_CHUNKS,)),
            pltpu.SemaphoreType.DMA((N_CHUNKS,)),
            pltpu.SemaphoreType.DMA((N_CHUNKS,)),
        ],
        compiler_params=pltpu.CompilerParams(collective_id=0),
    )(x)
    return out


# device time: 239471 ns/iter; 1.8221x vs baseline; 1.0568x over previous
import jax
import jax.numpy as jnp
from jax import lax
from jax.experimental import pallas as pl
from jax.experimental.pallas import tpu as pltpu

N_CHUNKS = 16


def kernel(x):
    x = x.astype(jnp.bfloat16)
    m, n = x.shape
    rows = m // N_CHUNKS

    def body(x_ref, out_ref, xbuf, vx, vp, vs,
             lsem_a, lsem_b, lsem_c, send_x, recv_x, send_y, recv_y):
        my_x = lax.axis_index("x")
        my_y = lax.axis_index("y")
        x_peer = (1 - my_x, my_y)
        y_peer = (my_x, 1 - my_y)

        barrier = pltpu.get_barrier_semaphore()
        for nbr in (x_peer, y_peer):
            pl.semaphore_signal(
                barrier, inc=1, device_id=nbr,
                device_id_type=pl.DeviceIdType.MESH,
            )
        pl.semaphore_wait(barrier, 2)

        x_rdmas = []
        for k in range(N_CHUNKS):
            r0 = k * rows
            r = pltpu.make_async_remote_copy(
                src_ref=x_ref.at[pl.ds(r0, rows), :],
                dst_ref=xbuf.at[pl.ds(r0, rows), :],
                send_sem=send_x.at[k],
                recv_sem=recv_x.at[k],
                device_id=x_peer,
                device_id_type=pl.DeviceIdType.MESH,
            )
            r.start()
            x_rdmas.append(r)

        y_rdmas = []
        store_cps = []
        for k in range(N_CHUNKS):
            r0 = k * rows
            x_rdmas[k].wait_recv()
            cp_x = pltpu.make_async_copy(
                x_ref.at[pl.ds(r0, rows), :], vx, lsem_a)
            cp_p = pltpu.make_async_copy(
                xbuf.at[pl.ds(r0, rows), :], vp, lsem_b)
            cp_x.start()
            cp_p.start()
            cp_x.wait()
            cp_p.wait()
            vs[k, :, :] = vx[:, :] + vp[:, :]
            ry = pltpu.make_async_remote_copy(
                src_ref=vs.at[k],
                dst_ref=out_ref.at[pl.ds(r0, rows), pl.ds(my_y * n, n)],
                send_sem=send_y.at[k],
                recv_sem=recv_y.at[k],
                device_id=y_peer,
                device_id_type=pl.DeviceIdType.MESH,
            )
            ry.start()
            y_rdmas.append(ry)
            cp_o = pltpu.make_async_copy(
                vs.at[k],
                out_ref.at[pl.ds(r0, rows), pl.ds(my_y * n, n)],
                lsem_c.at[k],
            )
            cp_o.start()
            store_cps.append(cp_o)

        for k in range(N_CHUNKS):
            x_rdmas[k].wait_send()
            y_rdmas[k].wait_send()
            y_rdmas[k].wait_recv()
            store_cps[k].wait()

    out, _ = pl.pallas_call(
        body,
        out_shape=[
            jax.ShapeDtypeStruct((m, 2 * n), jnp.bfloat16),
            jax.ShapeDtypeStruct((m, n), jnp.bfloat16),
        ],
        in_specs=[pl.BlockSpec(memory_space=pl.ANY)],
        out_specs=[
            pl.BlockSpec(memory_space=pl.ANY),
            pl.BlockSpec(memory_space=pl.ANY),
        ],
        scratch_shapes=[
            pltpu.VMEM((rows, n), jnp.bfloat16),
            pltpu.VMEM((rows, n), jnp.bfloat16),
            pltpu.VMEM((N_CHUNKS, rows, n), jnp.bfloat16),
            pltpu.SemaphoreType.DMA,
            pltpu.SemaphoreType.DMA,
            pltpu.SemaphoreType.DMA((N_CHUNKS,)),
            pltpu.SemaphoreType.DMA((N_CHUNKS,)),
            pltpu.SemaphoreType.DMA((N_CHUNKS,)),
            pltpu.SemaphoreType.DMA((N_CHUNKS,)),
            pltpu.SemaphoreType.DMA((N_CHUNKS,)),
        ],
        compiler_params=pltpu.CompilerParams(collective_id=0),
    )(x)
    return out


# device time: 231610 ns/iter; 1.8839x vs baseline; 1.0339x over previous
import jax
import jax.numpy as jnp
from jax import lax
from jax.experimental import pallas as pl
from jax.experimental.pallas import tpu as pltpu

N_CHUNKS = 16
PREF = 4


def kernel(x):
    x = x.astype(jnp.bfloat16)
    m, n = x.shape
    rows = m // N_CHUNKS

    def body(x_ref, out_ref, xrecv, vx,
             lsem_x, lsem_o, send_x, recv_x, send_y, recv_y):
        my_x = lax.axis_index("x")
        my_y = lax.axis_index("y")
        x_peer = (1 - my_x, my_y)
        y_peer = (my_x, 1 - my_y)

        barrier = pltpu.get_barrier_semaphore()
        for nbr in (x_peer, y_peer):
            pl.semaphore_signal(
                barrier, inc=1, device_id=nbr,
                device_id_type=pl.DeviceIdType.MESH,
            )
        pl.semaphore_wait(barrier, 2)

        x_rdmas = []
        for k in range(N_CHUNKS):
            r = pltpu.make_async_remote_copy(
                src_ref=x_ref.at[pl.ds(k * rows, rows), :],
                dst_ref=xrecv.at[k],
                send_sem=send_x.at[k],
                recv_sem=recv_x.at[k],
                device_id=x_peer,
                device_id_type=pl.DeviceIdType.MESH,
            )
            r.start()
            x_rdmas.append(r)

        def load_mine(k):
            cp = pltpu.make_async_copy(
                x_ref.at[pl.ds(k * rows, rows), :],
                vx.at[k % PREF],
                lsem_x.at[k % PREF],
            )
            cp.start()
            return cp

        loads = {}
        for k in range(PREF):
            loads[k] = load_mine(k)

        y_rdmas = []
        store_cps = []
        for k in range(N_CHUNKS):
            x_rdmas[k].wait_recv()
            loads[k].wait()
            xrecv[k, :, :] = xrecv[k, :, :] + vx[k % PREF, :, :]
            ry = pltpu.make_async_remote_copy(
                src_ref=xrecv.at[k],
                dst_ref=out_ref.at[pl.ds(k * rows, rows), pl.ds(my_y * n, n)],
                send_sem=send_y.at[k],
                recv_sem=recv_y.at[k],
                device_id=y_peer,
                device_id_type=pl.DeviceIdType.MESH,
            )
            ry.start()
            y_rdmas.append(ry)
            cp_o = pltpu.make_async_copy(
                xrecv.at[k],
                out_ref.at[pl.ds(k * rows, rows), pl.ds(my_y * n, n)],
                lsem_o.at[k],
            )
            cp_o.start()
            store_cps.append(cp_o)
            if k + PREF < N_CHUNKS:
                loads[k + PREF] = load_mine(k + PREF)

        for k in range(N_CHUNKS):
            x_rdmas[k].wait_send()
            y_rdmas[k].wait_send()
            y_rdmas[k].wait_recv()
            store_cps[k].wait()

    return pl.pallas_call(
        body,
        out_shape=jax.ShapeDtypeStruct((m, 2 * n), jnp.bfloat16),
        in_specs=[pl.BlockSpec(memory_space=pl.ANY)],
        out_specs=pl.BlockSpec(memory_space=pl.ANY),
        scratch_shapes=[
            pltpu.VMEM((N_CHUNKS, rows, n), jnp.bfloat16),
            pltpu.VMEM((PREF, rows, n), jnp.bfloat16),
            pltpu.SemaphoreType.DMA((PREF,)),
            pltpu.SemaphoreType.DMA((N_CHUNKS,)),
            pltpu.SemaphoreType.DMA((N_CHUNKS,)),
            pltpu.SemaphoreType.DMA((N_CHUNKS,)),
            pltpu.SemaphoreType.DMA((N_CHUNKS,)),
            pltpu.SemaphoreType.DMA((N_CHUNKS,)),
        ],
        compiler_params=pltpu.CompilerParams(collective_id=0),
    )(x)
